# baseline (speedup 1.0000x reference)
import jax
import jax.numpy as jnp
from jax.experimental import pallas as pl
from jax.experimental.pallas import tpu as pltpu

_NUM_HEADS = 16
_BATCH = 8
_SEQ = 512
_SP = _SEQ + 1
_HB = 16


def _bias_kernel(w_ref, vb_ref, adj_ref, out_ref):
    h0 = pl.program_id(1) * _HB
    adj = adj_ref[0].astype(jnp.float32)
    row = jax.lax.broadcasted_iota(jnp.int32, (_SP, _SP), 0)
    col = jax.lax.broadcasted_iota(jnp.int32, (_SP, _SP), 1)
    is_border = (row == 0) | (col == 0)
    border_count = (row == 0).astype(jnp.float32) + (col == 0).astype(jnp.float32)
    for i in range(_HB):
        h = h0 + i
        w0 = w_ref[0, h]
        w1 = w_ref[1, h]
        vb = vb_ref[h]
        interior = w0 + adj * (w1 - w0)
        out_ref[0, i] = jnp.where(is_border, border_count * vb, interior)


def kernel(adj_matrix, emb_table, virtual_bias):
    adj = adj_matrix.astype(jnp.int32)
    adj_p = jnp.pad(adj, ((0, 0), (1, 0), (1, 0)))
    vb = virtual_bias.reshape(_NUM_HEADS)
    return pl.pallas_call(
        _bias_kernel,
        grid=(_BATCH, _NUM_HEADS // _HB),
        in_specs=[
            pl.BlockSpec(memory_space=pltpu.SMEM),
            pl.BlockSpec(memory_space=pltpu.SMEM),
            pl.BlockSpec((1, _SP, _SP), lambda b, h: (b, 0, 0)),
        ],
        out_specs=pl.BlockSpec((1, _HB, _SP, _SP), lambda b, h: (b, h, 0, 0)),
        out_shape=jax.ShapeDtypeStruct((_BATCH, _NUM_HEADS, _SP, _SP), jnp.float32),
        compiler_params=pltpu.CompilerParams(
            dimension_semantics=("parallel", "parallel"),
        ),
    )(emb_table, vb, adj_p)


# speedup vs baseline: 1.0691x; 1.0691x over previous
import jax
import jax.numpy as jnp
from jax.experimental import pallas as pl
from jax.experimental.pallas import tpu as pltpu

_NUM_HEADS = 16
_BATCH = 8
_SEQ = 512
_SP = _SEQ + 1
_HB = 8


def _bias_kernel(w_ref, vb_ref, adj_ref, out_ref):
    h0 = pl.program_id(1) * _HB
    adj = adj_ref[0].astype(jnp.float32)
    adj = jnp.concatenate([jnp.zeros((1, _SEQ), jnp.float32), adj], axis=0)
    adj = jnp.concatenate([jnp.zeros((_SP, 1), jnp.float32), adj], axis=1)
    row = jax.lax.broadcasted_iota(jnp.int32, (_SP, _SP), 0)
    col = jax.lax.broadcasted_iota(jnp.int32, (_SP, _SP), 1)
    is_border = (row == 0) | (col == 0)
    border_count = (row == 0).astype(jnp.float32) + (col == 0).astype(jnp.float32)
    for i in range(_HB):
        h = h0 + i
        w0 = w_ref[0, h]
        w1 = w_ref[1, h]
        vb = vb_ref[h]
        interior = w0 + adj * (w1 - w0)
        out_ref[0, i] = jnp.where(is_border, border_count * vb, interior)


def kernel(adj_matrix, emb_table, virtual_bias):
    adj = adj_matrix.astype(jnp.int32)
    vb = virtual_bias.reshape(_NUM_HEADS)
    return pl.pallas_call(
        _bias_kernel,
        grid=(_BATCH, _NUM_HEADS // _HB),
        in_specs=[
            pl.BlockSpec(memory_space=pltpu.SMEM),
            pl.BlockSpec(memory_space=pltpu.SMEM),
            pl.BlockSpec((1, _SEQ, _SEQ), lambda b, h: (b, 0, 0)),
        ],
        out_specs=pl.BlockSpec((1, _HB, _SP, _SP), lambda b, h: (b, h, 0, 0)),
        out_shape=jax.ShapeDtypeStruct((_BATCH, _NUM_HEADS, _SP, _SP), jnp.float32),
        compiler_params=pltpu.CompilerParams(
            dimension_semantics=("parallel", "parallel"),
        ),
    )(emb_table, vb, adj)
